# baseline (speedup 1.0000x reference)
import functools

import jax
import jax.numpy as jnp
from jax import lax
from jax.experimental import pallas as pl
from jax.experimental.pallas import tpu as pltpu
from jax.experimental.pallas import tpu_sc as plsc

B = 8
N = 4096
M = 512
K = 32
R2 = 0.2 ** 2
D_PTS = 64
C_IN = 67
C_PAD = 128
P_ROWS = B * M * K
EPS = 1e-5

NW = 32


def _fps_body(xyzT_ref, out_ref):
    x = xyzT_ref[0]
    y = xyzT_ref[1]
    z = xyzT_ref[2]
    iota_n = lax.broadcasted_iota(jnp.int32, (B, N), 1)
    iota_m = lax.broadcasted_iota(jnp.int32, (B, M), 1)

    def body(i, st):
        dist, far, nx, ny, nz = st
        m = iota_n == far
        cx = jnp.sum(jnp.where(m, x, 0.0), axis=1, keepdims=True)
        cy = jnp.sum(jnp.where(m, y, 0.0), axis=1, keepdims=True)
        cz = jnp.sum(jnp.where(m, z, 0.0), axis=1, keepdims=True)
        sel = iota_m == i
        nx = jnp.where(sel, cx, nx)
        ny = jnp.where(sel, cy, ny)
        nz = jnp.where(sel, cz, nz)
        dx = x - cx
        dy = y - cy
        dz = z - cz
        d = (dx * dx + dy * dy) + dz * dz
        dist = jnp.minimum(dist, d)
        mx = jnp.max(dist, axis=1, keepdims=True)
        far = jnp.min(jnp.where(dist == mx, iota_n, N), axis=1, keepdims=True)
        return dist, far, nx, ny, nz

    dist0 = jnp.full((B, N), 1e10, jnp.float32)
    far0 = jnp.zeros((B, 1), jnp.int32)
    zM = jnp.zeros((B, M), jnp.float32)
    _, _, nx, ny, nz = lax.fori_loop(0, M, body, (dist0, far0, zM, zM, zM))
    out_ref[0] = nx
    out_ref[1] = ny
    out_ref[2] = nz


def _run_fps(xyzT):
    return pl.pallas_call(
        _fps_body,
        out_shape=jax.ShapeDtypeStruct((3, B, M), jnp.float32),
    )(xyzT)


TILE = 512
NTILE = N // TILE


def _ballq_body(nx_ref, px_ref, out_ref, carry_ref, acc_ref):
    b = pl.program_id(0)
    j = pl.program_id(1)
    n8 = nx_ref[0]
    p8 = px_ref[0]
    mm = lax.dot_general(n8, p8, (((1,), (0,)), ((), ())),
                         preferred_element_type=jnp.float32)
    n2 = jnp.sum(n8 * n8, axis=1, keepdims=True)
    p2 = jnp.sum(p8 * p8, axis=0, keepdims=True)
    d = (-2.0 * mm + n2) + p2
    m = jnp.where(d <= R2, 1.0, 0.0)
    r1 = lax.broadcasted_iota(jnp.int32, (TILE, TILE), 0)
    r2 = lax.broadcasted_iota(jnp.int32, (TILE, TILE), 1)
    lt = jnp.where(r1 <= r2, 1.0, 0.0)

    @pl.when(j == 0)
    def _():
        carry_ref[...] = jnp.zeros_like(carry_ref)
        acc_ref[...] = jnp.zeros_like(acc_ref)
        out_ref[...] = jnp.zeros_like(out_ref)

    rank = lax.dot_general(m, lt, (((1,), (0,)), ((), ())))
    rank = rank + carry_ref[:, 0:1]
    carry_ref[:, 0:1] = rank[:, TILE - 1:TILE]
    rcb = jnp.minimum(rank, 40.0).astype(jnp.bfloat16)
    one = jnp.ones((), jnp.bfloat16)
    zero = jnp.zeros((), jnp.bfloat16)
    half = TILE // 2
    cols = []
    for s in range(K):
        ind = jnp.where(rcb <= jnp.bfloat16(s), one, zero)
        c0 = jnp.sum(ind[:, :half], axis=1, keepdims=True).astype(jnp.float32)
        c1 = jnp.sum(ind[:, half:], axis=1, keepdims=True).astype(jnp.float32)
        cols.append(c0 + c1)
    acc_ref[...] += jnp.concatenate(cols, axis=1)

    @pl.when(j == NTILE - 1)
    def _():
        idx = acc_ref[...].astype(jnp.int32)
        first = idx[:, 0:1]
        idx = jnp.where(idx == N, first, idx)
        out_ref[...] = idx + b * N


def _run_ballq(nxyz8, xyzB8):
    return pl.pallas_call(
        _ballq_body,
        grid=(B, NTILE),
        in_specs=[
            pl.BlockSpec((1, M, 8), lambda b, j: (b, 0, 0)),
            pl.BlockSpec((1, 8, TILE), lambda b, j: (b, 0, j)),
        ],
        out_specs=pl.BlockSpec((M, K), lambda b, j: (b, 0)),
        out_shape=jax.ShapeDtypeStruct((B * M, K), jnp.int32),
        scratch_shapes=[
            pltpu.VMEM((M, 128), jnp.float32),
            pltpu.VMEM((M, K), jnp.float32),
        ],
    )(nxyz8, xyzB8)


IDX_PER_W = P_ROWS // NW
GCHUNK = 512


def _gather_body(table_hbm, idx_hbm, out_hbm, idx_v, rows_v, sem):
    wid = lax.axis_index("s") * 2 + lax.axis_index("c")
    base = wid * IDX_PER_W

    def chunk(j, _):
        off = base + j * GCHUNK
        pltpu.sync_copy(idx_hbm.at[pl.ds(off, GCHUNK)], idx_v)
        pltpu.async_copy(table_hbm.at[idx_v], rows_v, sem).wait()
        pltpu.sync_copy(rows_v, out_hbm.at[pl.ds(off, GCHUNK)])
        return 0

    lax.fori_loop(0, IDX_PER_W // GCHUNK, chunk, 0)


def _run_gather(table, idx_flat):
    mesh = plsc.VectorSubcoreMesh(core_axis_name="c", subcore_axis_name="s")
    f = functools.partial(
        pl.kernel,
        mesh=mesh,
        out_type=jax.ShapeDtypeStruct((P_ROWS, C_PAD), jnp.float32),
        scratch_types=[
            pltpu.VMEM((GCHUNK,), jnp.int32),
            pltpu.VMEM((GCHUNK, C_PAD), jnp.float32),
            pltpu.SemaphoreType.DMA,
        ],
    )(_gather_body)
    return f(table, idx_flat)


MCHUNK = 64
GRID_MLP = (B * M) // MCHUNK
RCHUNK = MCHUNK * K


def _l0_body(rows_ref, nx_ref, w_ref, y_ref, st_ref):
    x = rows_ref[...] - nx_ref[...][:, None, :]
    xf = x.reshape(RCHUNK, C_PAD)
    y = lax.dot_general(xf, w_ref[...], (((1,), (0,)), ((), ())))
    y_ref[...] = y
    @pl.when(pl.program_id(0) == 0)
    def _():
        st_ref[...] = jnp.zeros_like(st_ref)
    s = jnp.sum(y, axis=0, keepdims=True)
    sq = jnp.sum(y * y, axis=0, keepdims=True)
    st_ref[0:1, :] += s
    st_ref[1:2, :] += sq


def _run_l0(rows3d, nxpad, w0t):
    return pl.pallas_call(
        _l0_body,
        grid=(GRID_MLP,),
        in_specs=[
            pl.BlockSpec((MCHUNK, K, C_PAD), lambda i: (i, 0, 0)),
            pl.BlockSpec((MCHUNK, C_PAD), lambda i: (i, 0)),
            pl.BlockSpec((C_PAD, 64), lambda i: (0, 0)),
        ],
        out_specs=[
            pl.BlockSpec((RCHUNK, 64), lambda i: (i, 0)),
            pl.BlockSpec((2, 64), lambda i: (0, 0)),
        ],
        out_shape=[
            jax.ShapeDtypeStruct((P_ROWS, 64), jnp.float32),
            jax.ShapeDtypeStruct((2, 64), jnp.float32),
        ],
    )(rows3d, nxpad, w0t)


def _mid_body(y_ref, st_ref, g_ref, b_ref, w_ref, o_ref, so_ref):
    inv_p = 1.0 / P_ROWS
    mean = st_ref[0:1, :] * inv_p
    var = st_ref[1:2, :] * inv_p - mean * mean
    a = g_ref[...] * lax.rsqrt(var + EPS)
    d = b_ref[...] - mean * a
    z = jnp.maximum(y_ref[...] * a + d, 0.0)
    y = lax.dot_general(z, w_ref[...], (((1,), (0,)), ((), ())))
    o_ref[...] = y
    @pl.when(pl.program_id(0) == 0)
    def _():
        so_ref[...] = jnp.zeros_like(so_ref)
    so_ref[0:1, :] += jnp.sum(y, axis=0, keepdims=True)
    so_ref[1:2, :] += jnp.sum(y * y, axis=0, keepdims=True)


def _run_mid(y_prev, stats, g, b, wt, c_out):
    c_in = y_prev.shape[1]
    return pl.pallas_call(
        _mid_body,
        grid=(GRID_MLP,),
        in_specs=[
            pl.BlockSpec((RCHUNK, c_in), lambda i: (i, 0)),
            pl.BlockSpec((2, c_in), lambda i: (0, 0)),
            pl.BlockSpec((1, c_in), lambda i: (0, 0)),
            pl.BlockSpec((1, c_in), lambda i: (0, 0)),
            pl.BlockSpec((c_in, c_out), lambda i: (0, 0)),
        ],
        out_specs=[
            pl.BlockSpec((RCHUNK, c_out), lambda i: (i, 0)),
            pl.BlockSpec((2, c_out), lambda i: (0, 0)),
        ],
        out_shape=[
            jax.ShapeDtypeStruct((P_ROWS, c_out), jnp.float32),
            jax.ShapeDtypeStruct((2, c_out), jnp.float32),
        ],
    )(y_prev, stats, g, b, wt)


def _l2_body(y_ref, st_ref, g_ref, b_ref, w_ref, mx_ref, mn_ref, so_ref):
    inv_p = 1.0 / P_ROWS
    mean = st_ref[0:1, :] * inv_p
    var = st_ref[1:2, :] * inv_p - mean * mean
    a = g_ref[...] * lax.rsqrt(var + EPS)
    d = b_ref[...] - mean * a
    z = jnp.maximum(y_ref[...] * a + d, 0.0)
    y = lax.dot_general(z, w_ref[...], (((1,), (0,)), ((), ())))
    y3 = y.reshape(MCHUNK, K, 128)
    mx_ref[...] = jnp.max(y3, axis=1)
    mn_ref[...] = jnp.min(y3, axis=1)
    @pl.when(pl.program_id(0) == 0)
    def _():
        so_ref[...] = jnp.zeros_like(so_ref)
    so_ref[0:1, :] += jnp.sum(y, axis=0, keepdims=True)
    so_ref[1:2, :] += jnp.sum(y * y, axis=0, keepdims=True)


def _run_l2(y1, stats, g, b, wt):
    return pl.pallas_call(
        _l2_body,
        grid=(GRID_MLP,),
        in_specs=[
            pl.BlockSpec((RCHUNK, 64), lambda i: (i, 0)),
            pl.BlockSpec((2, 64), lambda i: (0, 0)),
            pl.BlockSpec((1, 64), lambda i: (0, 0)),
            pl.BlockSpec((1, 64), lambda i: (0, 0)),
            pl.BlockSpec((64, 128), lambda i: (0, 0)),
        ],
        out_specs=[
            pl.BlockSpec((MCHUNK, 128), lambda i: (i, 0)),
            pl.BlockSpec((MCHUNK, 128), lambda i: (i, 0)),
            pl.BlockSpec((2, 128), lambda i: (0, 0)),
        ],
        out_shape=[
            jax.ShapeDtypeStruct((B * M, 128), jnp.float32),
            jax.ShapeDtypeStruct((B * M, 128), jnp.float32),
            jax.ShapeDtypeStruct((2, 128), jnp.float32),
        ],
    )(y1, stats, g, b, wt)


def _l3_body(mx_ref, mn_ref, st_ref, g_ref, b_ref, o_ref):
    inv_p = 1.0 / P_ROWS
    mean = st_ref[0:1, :] * inv_p
    var = st_ref[1:2, :] * inv_p - mean * mean
    a = g_ref[...] * lax.rsqrt(var + EPS)
    d = b_ref[...] - mean * a
    v = jnp.maximum(mx_ref[...] * a + d, mn_ref[...] * a + d)
    o_ref[...] = jnp.maximum(v, 0.0)


def _run_l3(ymax, ymin, stats, g, b):
    return pl.pallas_call(
        _l3_body,
        grid=(B,),
        in_specs=[
            pl.BlockSpec((M, 128), lambda i: (i, 0)),
            pl.BlockSpec((M, 128), lambda i: (i, 0)),
            pl.BlockSpec((2, 128), lambda i: (0, 0)),
            pl.BlockSpec((1, 128), lambda i: (0, 0)),
            pl.BlockSpec((1, 128), lambda i: (0, 0)),
        ],
        out_specs=pl.BlockSpec((M, 128), lambda i: (i, 0)),
        out_shape=jax.ShapeDtypeStruct((B * M, 128), jnp.float32),
    )(ymax, ymin, stats, g, b)


def kernel(xyz, points, W0, g0, b0, W1, g1, b1, W2, g2, b2):
    xyzT = jnp.transpose(xyz, (2, 0, 1))

    nxyzT = _run_fps(xyzT)
    new_xyz = jnp.transpose(nxyzT, (1, 2, 0))

    nxyz8 = jnp.pad(new_xyz, ((0, 0), (0, 0), (0, 5)))
    xyzB8 = jnp.pad(jnp.transpose(xyz, (0, 2, 1)),
                    ((0, 0), (0, 5), (0, 0)))
    idx_flat = _run_ballq(nxyz8, xyzB8).reshape(-1)

    table = jnp.concatenate(
        [points, xyz, jnp.zeros((B, N, C_PAD - C_IN), jnp.float32)], axis=-1
    ).reshape(B * N, C_PAD)
    rows = _run_gather(table, idx_flat)

    nxpad = jnp.concatenate(
        [jnp.zeros((B * M, D_PTS), jnp.float32),
         new_xyz.reshape(B * M, 3),
         jnp.zeros((B * M, C_PAD - C_IN), jnp.float32)], axis=-1)

    rows3d = rows.reshape(B * M, K, C_PAD)
    w0t = jnp.pad(jnp.transpose(W0), ((0, C_PAD - C_IN), (0, 0)))
    y0, st0 = _run_l0(rows3d, nxpad, w0t)
    y1, st1 = _run_mid(y0, st0, g0.reshape(1, 64), b0.reshape(1, 64),
                       jnp.transpose(W1), 64)
    ymax, ymin, st2 = _run_l2(y1, st1, g1.reshape(1, 64), b1.reshape(1, 64),
                              jnp.transpose(W2))
    pooled = _run_l3(ymax, ymin, st2, g2.reshape(1, 128), b2.reshape(1, 128))

    new_points = pooled.reshape(B, M, 128)
    return (new_xyz, new_points)


# speedup vs baseline: 1.3546x; 1.3546x over previous
import functools

import jax
import jax.numpy as jnp
from jax import lax
from jax.experimental import pallas as pl
from jax.experimental.pallas import tpu as pltpu
from jax.experimental.pallas import tpu_sc as plsc

B = 8
N = 4096
M = 512
K = 32
R2 = 0.2 ** 2
D_PTS = 64
C_IN = 67
C_PAD = 128
P_ROWS = B * M * K
EPS = 1e-5

NW = 32


def _fps_body(xyzT_ref, out_ref):
    x = xyzT_ref[0]
    y = xyzT_ref[1]
    z = xyzT_ref[2]
    iota_n = lax.broadcasted_iota(jnp.int32, (B, N), 1)
    iota_m = lax.broadcasted_iota(jnp.int32, (B, M), 1)

    def body(i, st):
        dist, far, nx, ny, nz = st
        m = iota_n == far
        cx = jnp.sum(jnp.where(m, x, 0.0), axis=1, keepdims=True)
        cy = jnp.sum(jnp.where(m, y, 0.0), axis=1, keepdims=True)
        cz = jnp.sum(jnp.where(m, z, 0.0), axis=1, keepdims=True)
        sel = iota_m == i
        nx = jnp.where(sel, cx, nx)
        ny = jnp.where(sel, cy, ny)
        nz = jnp.where(sel, cz, nz)
        dx = x - cx
        dy = y - cy
        dz = z - cz
        d = (dx * dx + dy * dy) + dz * dz
        dist = jnp.minimum(dist, d)
        mx = jnp.max(dist, axis=1, keepdims=True)
        far = jnp.min(jnp.where(dist == mx, iota_n, N), axis=1, keepdims=True)
        return dist, far, nx, ny, nz

    dist0 = jnp.full((B, N), 1e10, jnp.float32)
    far0 = jnp.zeros((B, 1), jnp.int32)
    zM = jnp.zeros((B, M), jnp.float32)
    _, _, nx, ny, nz = lax.fori_loop(0, M, body, (dist0, far0, zM, zM, zM))
    out_ref[0] = nx
    out_ref[1] = ny
    out_ref[2] = nz


def _run_fps(xyzT):
    return pl.pallas_call(
        _fps_body,
        out_shape=jax.ShapeDtypeStruct((3, B, M), jnp.float32),
    )(xyzT)


TILE = 512
NTILE = N // TILE


def _ballq_body(nx_ref, px_ref, out_ref, carry_ref, acc_ref):
    b = pl.program_id(0)
    j = pl.program_id(1)
    n8 = nx_ref[0]
    p8 = px_ref[0]
    mm = lax.dot_general(n8, p8, (((1,), (0,)), ((), ())),
                         preferred_element_type=jnp.float32)
    n2 = jnp.sum(n8 * n8, axis=1, keepdims=True)
    p2 = jnp.sum(p8 * p8, axis=0, keepdims=True)
    d = (-2.0 * mm + n2) + p2
    m = jnp.where(d <= R2, 1.0, 0.0)
    r1 = lax.broadcasted_iota(jnp.int32, (TILE, TILE), 0)
    r2 = lax.broadcasted_iota(jnp.int32, (TILE, TILE), 1)
    lt = jnp.where(r1 <= r2, 1.0, 0.0)

    @pl.when(j == 0)
    def _():
        carry_ref[...] = jnp.zeros_like(carry_ref)
        acc_ref[...] = jnp.zeros_like(acc_ref)
        out_ref[...] = jnp.zeros_like(out_ref)

    rank = lax.dot_general(m, lt, (((1,), (0,)), ((), ())))
    rank = rank + carry_ref[:, 0:1]
    carry_ref[:, 0:1] = rank[:, TILE - 1:TILE]
    cols = [jnp.sum(jnp.where(rank <= float(s), 1.0, 0.0), axis=1, keepdims=True)
            for s in range(K)]
    acc_ref[...] += jnp.concatenate(cols, axis=1)

    @pl.when(j == NTILE - 1)
    def _():
        idx = acc_ref[...].astype(jnp.int32)
        first = idx[:, 0:1]
        idx = jnp.where(idx == N, first, idx)
        out_ref[...] = idx + b * N


def _run_ballq(nxyz8, xyzB8):
    return pl.pallas_call(
        _ballq_body,
        grid=(B, NTILE),
        in_specs=[
            pl.BlockSpec((1, M, 8), lambda b, j: (b, 0, 0)),
            pl.BlockSpec((1, 8, TILE), lambda b, j: (b, 0, j)),
        ],
        out_specs=pl.BlockSpec((M, K), lambda b, j: (b, 0)),
        out_shape=jax.ShapeDtypeStruct((B * M, K), jnp.int32),
        scratch_shapes=[
            pltpu.VMEM((M, 128), jnp.float32),
            pltpu.VMEM((M, K), jnp.float32),
        ],
    )(nxyz8, xyzB8)


IDX_PER_W = P_ROWS // NW
GCHUNK = 512


def _gather_body(table_hbm, idx_hbm, out_hbm, idx_v, rows_v, sem):
    wid = lax.axis_index("s") * 2 + lax.axis_index("c")
    base = wid * IDX_PER_W

    def chunk(j, _):
        off = base + j * GCHUNK
        pltpu.sync_copy(idx_hbm.at[pl.ds(off, GCHUNK)], idx_v)
        pltpu.async_copy(table_hbm.at[idx_v], rows_v, sem).wait()
        pltpu.sync_copy(rows_v, out_hbm.at[pl.ds(off, GCHUNK)])
        return 0

    lax.fori_loop(0, IDX_PER_W // GCHUNK, chunk, 0)


def _run_gather(table, idx_flat):
    mesh = plsc.VectorSubcoreMesh(core_axis_name="c", subcore_axis_name="s")
    f = functools.partial(
        pl.kernel,
        mesh=mesh,
        out_type=jax.ShapeDtypeStruct((P_ROWS, C_PAD), jnp.float32),
        scratch_types=[
            pltpu.VMEM((GCHUNK,), jnp.int32),
            pltpu.VMEM((GCHUNK, C_PAD), jnp.float32),
            pltpu.SemaphoreType.DMA,
        ],
    )(_gather_body)
    return f(table, idx_flat)


MCHUNK = 64
GRID_MLP = (B * M) // MCHUNK
RCHUNK = MCHUNK * K


MC2 = 2 * MCHUNK
GRID2 = (B * M) // MC2


def _mlp_body(rows_ref, nx_ref, w0_ref, w1_ref, w2_ref, g0_ref, b0_ref,
              g1_ref, b1_ref, mx_ref, mn_ref, st2_ref, s_ref, st0_ref,
              st1_ref):
    p = pl.program_id(0)
    i = pl.program_id(1)
    inv_p = 1.0 / P_ROWS

    @pl.when(jnp.logical_and(p == 0, i == 0))
    def _():
        st0_ref[...] = jnp.zeros_like(st0_ref)
        st1_ref[...] = jnp.zeros_like(st1_ref)
        st2_ref[...] = jnp.zeros_like(st2_ref)

    @pl.when(p == 0)
    def _():
        x = rows_ref[...] - nx_ref[...][:, None, :]
        xf = x.reshape(2, RCHUNK, C_PAD)
        x2 = jnp.concatenate([xf[0], xf[1]], axis=1)
        y = lax.dot_general(x2, w0_ref[...], (((1,), (0,)), ((), ())))
        s_ref[pl.ds(i * RCHUNK, RCHUNK), :] = y
        ys = jnp.sum(y, axis=0, keepdims=True)
        yq = jnp.sum(y * y, axis=0, keepdims=True)
        st0_ref[0:1, :] += ys[:, :64] + ys[:, 64:]
        st0_ref[1:2, :] += yq[:, :64] + yq[:, 64:]

    @pl.when(p == 1)
    def _():
        y0 = s_ref[pl.ds(i * RCHUNK, RCHUNK), :]
        mean = st0_ref[0:1, :] * inv_p
        var = st0_ref[1:2, :] * inv_p - mean * mean
        a = g0_ref[...] * lax.rsqrt(var + EPS)
        d = b0_ref[...] - mean * a
        a2 = jnp.concatenate([a, a], axis=1)
        d2 = jnp.concatenate([d, d], axis=1)
        z = jnp.maximum(y0 * a2 + d2, 0.0)
        y = lax.dot_general(z, w1_ref[...], (((1,), (0,)), ((), ())))
        s_ref[pl.ds(i * RCHUNK, RCHUNK), :] = y
        ys = jnp.sum(y, axis=0, keepdims=True)
        yq = jnp.sum(y * y, axis=0, keepdims=True)
        st1_ref[0:1, :] += ys[:, :64] + ys[:, 64:]
        st1_ref[1:2, :] += yq[:, :64] + yq[:, 64:]

    @pl.when(p == 2)
    def _():
        y1 = s_ref[pl.ds(i * RCHUNK, RCHUNK), :]
        mean = st1_ref[0:1, :] * inv_p
        var = st1_ref[1:2, :] * inv_p - mean * mean
        a = g1_ref[...] * lax.rsqrt(var + EPS)
        d = b1_ref[...] - mean * a
        a2 = jnp.concatenate([a, a], axis=1)
        d2 = jnp.concatenate([d, d], axis=1)
        z = jnp.maximum(y1 * a2 + d2, 0.0)
        y = lax.dot_general(z, w2_ref[...], (((1,), (0,)), ((), ())))
        ya = y[:, :128].reshape(MCHUNK, K, 128)
        yb = y[:, 128:].reshape(MCHUNK, K, 128)
        mx_ref[...] = jnp.concatenate(
            [jnp.max(ya, axis=1), jnp.max(yb, axis=1)], axis=0)
        mn_ref[...] = jnp.concatenate(
            [jnp.min(ya, axis=1), jnp.min(yb, axis=1)], axis=0)
        ys = jnp.sum(y, axis=0, keepdims=True)
        yq = jnp.sum(y * y, axis=0, keepdims=True)
        st2_ref[0:1, :] += ys[:, :128] + ys[:, 128:]
        st2_ref[1:2, :] += yq[:, :128] + yq[:, 128:]


def _run_mlp(rows3d, nxpad, w0t, w1t, w2t, g0, b0, g1, b1):
    z64 = jnp.zeros((C_PAD, 64), jnp.float32)
    w0d = jnp.concatenate(
        [jnp.concatenate([w0t, z64], axis=1),
         jnp.concatenate([z64, w0t], axis=1)], axis=0)
    z6 = jnp.zeros((64, 64), jnp.float32)
    w1d = jnp.concatenate(
        [jnp.concatenate([w1t, z6], axis=1),
         jnp.concatenate([z6, w1t], axis=1)], axis=0)
    z12 = jnp.zeros((64, 128), jnp.float32)
    w2d = jnp.concatenate(
        [jnp.concatenate([w2t, z12], axis=1),
         jnp.concatenate([z12, w2t], axis=1)], axis=0)
    return pl.pallas_call(
        _mlp_body,
        grid=(3, GRID2),
        in_specs=[
            pl.BlockSpec((MC2, K, C_PAD),
                         lambda p, i: (jnp.where(p == 0, i, 0), 0, 0)),
            pl.BlockSpec((MC2, C_PAD),
                         lambda p, i: (jnp.where(p == 0, i, 0), 0)),
            pl.BlockSpec((2 * C_PAD, 128), lambda p, i: (0, 0)),
            pl.BlockSpec((128, 128), lambda p, i: (0, 0)),
            pl.BlockSpec((128, 256), lambda p, i: (0, 0)),
            pl.BlockSpec((1, 64), lambda p, i: (0, 0)),
            pl.BlockSpec((1, 64), lambda p, i: (0, 0)),
            pl.BlockSpec((1, 64), lambda p, i: (0, 0)),
            pl.BlockSpec((1, 64), lambda p, i: (0, 0)),
        ],
        out_specs=[
            pl.BlockSpec((MC2, 128), lambda p, i: (i, 0)),
            pl.BlockSpec((MC2, 128), lambda p, i: (i, 0)),
            pl.BlockSpec((2, 128), lambda p, i: (0, 0)),
        ],
        out_shape=[
            jax.ShapeDtypeStruct((B * M, 128), jnp.float32),
            jax.ShapeDtypeStruct((B * M, 128), jnp.float32),
            jax.ShapeDtypeStruct((2, 128), jnp.float32),
        ],
        scratch_shapes=[
            pltpu.VMEM((P_ROWS // 2, 128), jnp.float32),
            pltpu.VMEM((2, 64), jnp.float32),
            pltpu.VMEM((2, 64), jnp.float32),
        ],
    )(rows3d, nxpad, w0d, w1d, w2d, g0, b0, g1, b1)


def _l3_body(mx_ref, mn_ref, st_ref, g_ref, b_ref, o_ref):
    inv_p = 1.0 / P_ROWS
    mean = st_ref[0:1, :] * inv_p
    var = st_ref[1:2, :] * inv_p - mean * mean
    a = g_ref[...] * lax.rsqrt(var + EPS)
    d = b_ref[...] - mean * a
    v = jnp.maximum(mx_ref[...] * a + d, mn_ref[...] * a + d)
    o_ref[...] = jnp.maximum(v, 0.0)


def _run_l3(ymax, ymin, stats, g, b):
    return pl.pallas_call(
        _l3_body,
        grid=(B,),
        in_specs=[
            pl.BlockSpec((M, 128), lambda i: (i, 0)),
            pl.BlockSpec((M, 128), lambda i: (i, 0)),
            pl.BlockSpec((2, 128), lambda i: (0, 0)),
            pl.BlockSpec((1, 128), lambda i: (0, 0)),
            pl.BlockSpec((1, 128), lambda i: (0, 0)),
        ],
        out_specs=pl.BlockSpec((M, 128), lambda i: (i, 0)),
        out_shape=jax.ShapeDtypeStruct((B * M, 128), jnp.float32),
    )(ymax, ymin, stats, g, b)


def kernel(xyz, points, W0, g0, b0, W1, g1, b1, W2, g2, b2):
    xyzT = jnp.transpose(xyz, (2, 0, 1))

    nxyzT = _run_fps(xyzT)
    new_xyz = jnp.transpose(nxyzT, (1, 2, 0))

    nxyz8 = jnp.pad(new_xyz, ((0, 0), (0, 0), (0, 5)))
    xyzB8 = jnp.pad(jnp.transpose(xyz, (0, 2, 1)),
                    ((0, 0), (0, 5), (0, 0)))
    idx_flat = _run_ballq(nxyz8, xyzB8).reshape(-1)

    table = jnp.concatenate(
        [points, xyz, jnp.zeros((B, N, C_PAD - C_IN), jnp.float32)], axis=-1
    ).reshape(B * N, C_PAD)
    rows = _run_gather(table, idx_flat)

    nxpad = jnp.concatenate(
        [jnp.zeros((B * M, D_PTS), jnp.float32),
         new_xyz.reshape(B * M, 3),
         jnp.zeros((B * M, C_PAD - C_IN), jnp.float32)], axis=-1)

    rows3d = rows.reshape(B * M, K, C_PAD)
    w0t = jnp.pad(jnp.transpose(W0), ((0, C_PAD - C_IN), (0, 0)))
    ymax, ymin, st2 = _run_mlp(rows3d, nxpad, w0t, jnp.transpose(W1),
                               jnp.transpose(W2), g0.reshape(1, 64),
                               b0.reshape(1, 64), g1.reshape(1, 64),
                               b1.reshape(1, 64))
    pooled = _run_l3(ymax, ymin, st2, g2.reshape(1, 128), b2.reshape(1, 128))

    new_points = pooled.reshape(B, M, 128)
    return (new_xyz, new_points)


# speedup vs baseline: 1.3588x; 1.0031x over previous
import functools

import jax
import jax.numpy as jnp
from jax import lax
from jax.experimental import pallas as pl
from jax.experimental.pallas import tpu as pltpu
from jax.experimental.pallas import tpu_sc as plsc

B = 8
N = 4096
M = 512
K = 32
R2 = 0.2 ** 2
D_PTS = 64
C_IN = 67
C_PAD = 128
P_ROWS = B * M * K
EPS = 1e-5

NW = 32


def _fps_body(xyzT_ref, out_ref):
    x = xyzT_ref[0]
    y = xyzT_ref[1]
    z = xyzT_ref[2]
    iota_n = lax.broadcasted_iota(jnp.int32, (B, N), 1)
    iota_m = lax.broadcasted_iota(jnp.int32, (B, M), 1)

    def body(i, st):
        dist, far, nx, ny, nz = st
        m = iota_n == far
        cx = jnp.sum(jnp.where(m, x, 0.0), axis=1, keepdims=True)
        cy = jnp.sum(jnp.where(m, y, 0.0), axis=1, keepdims=True)
        cz = jnp.sum(jnp.where(m, z, 0.0), axis=1, keepdims=True)
        sel = iota_m == i
        nx = jnp.where(sel, cx, nx)
        ny = jnp.where(sel, cy, ny)
        nz = jnp.where(sel, cz, nz)
        dx = x - cx
        dy = y - cy
        dz = z - cz
        d = (dx * dx + dy * dy) + dz * dz
        dist = jnp.minimum(dist, d)
        mx = jnp.max(dist, axis=1, keepdims=True)
        far = jnp.min(jnp.where(dist == mx, iota_n, N), axis=1, keepdims=True)
        return dist, far, nx, ny, nz

    dist0 = jnp.full((B, N), 1e10, jnp.float32)
    far0 = jnp.zeros((B, 1), jnp.int32)
    zM = jnp.zeros((B, M), jnp.float32)
    _, _, nx, ny, nz = lax.fori_loop(0, M, body, (dist0, far0, zM, zM, zM),
                                     unroll=4)
    out_ref[0] = nx
    out_ref[1] = ny
    out_ref[2] = nz


def _run_fps(xyzT):
    return pl.pallas_call(
        _fps_body,
        out_shape=jax.ShapeDtypeStruct((3, B, M), jnp.float32),
    )(xyzT)


TILE = 512
NTILE = N // TILE


def _ballq_body(nx_ref, px_ref, out_ref, carry_ref, acc_ref):
    b = pl.program_id(0)
    j = pl.program_id(1)
    n8 = nx_ref[0]
    p8 = px_ref[0]
    mm = lax.dot_general(n8, p8, (((1,), (0,)), ((), ())),
                         preferred_element_type=jnp.float32)
    n2 = jnp.sum(n8 * n8, axis=1, keepdims=True)
    p2 = jnp.sum(p8 * p8, axis=0, keepdims=True)
    d = (-2.0 * mm + n2) + p2
    m = jnp.where(d <= R2, 1.0, 0.0)
    r1 = lax.broadcasted_iota(jnp.int32, (TILE, TILE), 0)
    r2 = lax.broadcasted_iota(jnp.int32, (TILE, TILE), 1)
    lt = jnp.where(r1 <= r2, 1.0, 0.0)

    @pl.when(j == 0)
    def _():
        carry_ref[...] = jnp.zeros_like(carry_ref)
        acc_ref[...] = jnp.zeros_like(acc_ref)
        out_ref[...] = jnp.zeros_like(out_ref)

    rank = lax.dot_general(m, lt, (((1,), (0,)), ((), ())))
    rank = rank + carry_ref[:, 0:1]
    carry_ref[:, 0:1] = rank[:, TILE - 1:TILE]
    cols = [jnp.sum(jnp.where(rank <= float(s), 1.0, 0.0), axis=1, keepdims=True)
            for s in range(K)]
    acc_ref[...] += jnp.concatenate(cols, axis=1)

    @pl.when(j == NTILE - 1)
    def _():
        idx = acc_ref[...].astype(jnp.int32)
        first = idx[:, 0:1]
        idx = jnp.where(idx == N, first, idx)
        out_ref[...] = idx + b * N


def _run_ballq(nxyz8, xyzB8):
    return pl.pallas_call(
        _ballq_body,
        grid=(B, NTILE),
        in_specs=[
            pl.BlockSpec((1, M, 8), lambda b, j: (b, 0, 0)),
            pl.BlockSpec((1, 8, TILE), lambda b, j: (b, 0, j)),
        ],
        out_specs=pl.BlockSpec((M, K), lambda b, j: (b, 0)),
        out_shape=jax.ShapeDtypeStruct((B * M, K), jnp.int32),
        scratch_shapes=[
            pltpu.VMEM((M, 128), jnp.float32),
            pltpu.VMEM((M, K), jnp.float32),
        ],
    )(nxyz8, xyzB8)


IDX_PER_W = P_ROWS // NW
GCHUNK = 512


def _gather_body(table_hbm, idx_hbm, out_hbm, idx_v, rows_v, sem):
    wid = lax.axis_index("s") * 2 + lax.axis_index("c")
    base = wid * IDX_PER_W

    def chunk(j, _):
        off = base + j * GCHUNK
        pltpu.sync_copy(idx_hbm.at[pl.ds(off, GCHUNK)], idx_v)
        pltpu.async_copy(table_hbm.at[idx_v], rows_v, sem).wait()
        pltpu.sync_copy(rows_v, out_hbm.at[pl.ds(off, GCHUNK)])
        return 0

    lax.fori_loop(0, IDX_PER_W // GCHUNK, chunk, 0)


def _run_gather(table, idx_flat):
    mesh = plsc.VectorSubcoreMesh(core_axis_name="c", subcore_axis_name="s")
    f = functools.partial(
        pl.kernel,
        mesh=mesh,
        out_type=jax.ShapeDtypeStruct((P_ROWS, C_PAD), jnp.float32),
        scratch_types=[
            pltpu.VMEM((GCHUNK,), jnp.int32),
            pltpu.VMEM((GCHUNK, C_PAD), jnp.float32),
            pltpu.SemaphoreType.DMA,
        ],
    )(_gather_body)
    return f(table, idx_flat)


MCHUNK = 64
GRID_MLP = (B * M) // MCHUNK
RCHUNK = MCHUNK * K


MC2 = 2 * MCHUNK
GRID2 = (B * M) // MC2


def _mlp_body(rows_ref, nx_ref, w0_ref, w1_ref, w2_ref, g0_ref, b0_ref,
              g1_ref, b1_ref, mx_ref, mn_ref, st2_ref, s_ref, st0_ref,
              st1_ref):
    p = pl.program_id(0)
    i = pl.program_id(1)
    inv_p = 1.0 / P_ROWS

    @pl.when(jnp.logical_and(p == 0, i == 0))
    def _():
        st0_ref[...] = jnp.zeros_like(st0_ref)
        st1_ref[...] = jnp.zeros_like(st1_ref)
        st2_ref[...] = jnp.zeros_like(st2_ref)

    @pl.when(p == 0)
    def _():
        x = rows_ref[...] - nx_ref[...][:, None, :]
        xf = x.reshape(2, RCHUNK, C_PAD)
        x2 = jnp.concatenate([xf[0], xf[1]], axis=1)
        y = lax.dot_general(x2, w0_ref[...], (((1,), (0,)), ((), ())))
        s_ref[pl.ds(i * RCHUNK, RCHUNK), :] = y
        ys = jnp.sum(y, axis=0, keepdims=True)
        yq = jnp.sum(y * y, axis=0, keepdims=True)
        st0_ref[0:1, :] += ys[:, :64] + ys[:, 64:]
        st0_ref[1:2, :] += yq[:, :64] + yq[:, 64:]

    @pl.when(p == 1)
    def _():
        y0 = s_ref[pl.ds(i * RCHUNK, RCHUNK), :]
        mean = st0_ref[0:1, :] * inv_p
        var = st0_ref[1:2, :] * inv_p - mean * mean
        a = g0_ref[...] * lax.rsqrt(var + EPS)
        d = b0_ref[...] - mean * a
        a2 = jnp.concatenate([a, a], axis=1)
        d2 = jnp.concatenate([d, d], axis=1)
        z = jnp.maximum(y0 * a2 + d2, 0.0)
        y = lax.dot_general(z, w1_ref[...], (((1,), (0,)), ((), ())))
        s_ref[pl.ds(i * RCHUNK, RCHUNK), :] = y
        ys = jnp.sum(y, axis=0, keepdims=True)
        yq = jnp.sum(y * y, axis=0, keepdims=True)
        st1_ref[0:1, :] += ys[:, :64] + ys[:, 64:]
        st1_ref[1:2, :] += yq[:, :64] + yq[:, 64:]

    @pl.when(p == 2)
    def _():
        y1 = s_ref[pl.ds(i * RCHUNK, RCHUNK), :]
        mean = st1_ref[0:1, :] * inv_p
        var = st1_ref[1:2, :] * inv_p - mean * mean
        a = g1_ref[...] * lax.rsqrt(var + EPS)
        d = b1_ref[...] - mean * a
        a2 = jnp.concatenate([a, a], axis=1)
        d2 = jnp.concatenate([d, d], axis=1)
        z = jnp.maximum(y1 * a2 + d2, 0.0)
        y = lax.dot_general(z, w2_ref[...], (((1,), (0,)), ((), ())))
        ya = y[:, :128].reshape(MCHUNK, K, 128)
        yb = y[:, 128:].reshape(MCHUNK, K, 128)
        mx_ref[...] = jnp.concatenate(
            [jnp.max(ya, axis=1), jnp.max(yb, axis=1)], axis=0)
        mn_ref[...] = jnp.concatenate(
            [jnp.min(ya, axis=1), jnp.min(yb, axis=1)], axis=0)
        ys = jnp.sum(y, axis=0, keepdims=True)
        yq = jnp.sum(y * y, axis=0, keepdims=True)
        st2_ref[0:1, :] += ys[:, :128] + ys[:, 128:]
        st2_ref[1:2, :] += yq[:, :128] + yq[:, 128:]


def _run_mlp(rows3d, nxpad, w0t, w1t, w2t, g0, b0, g1, b1):
    z64 = jnp.zeros((C_PAD, 64), jnp.float32)
    w0d = jnp.concatenate(
        [jnp.concatenate([w0t, z64], axis=1),
         jnp.concatenate([z64, w0t], axis=1)], axis=0)
    z6 = jnp.zeros((64, 64), jnp.float32)
    w1d = jnp.concatenate(
        [jnp.concatenate([w1t, z6], axis=1),
         jnp.concatenate([z6, w1t], axis=1)], axis=0)
    z12 = jnp.zeros((64, 128), jnp.float32)
    w2d = jnp.concatenate(
        [jnp.concatenate([w2t, z12], axis=1),
         jnp.concatenate([z12, w2t], axis=1)], axis=0)
    return pl.pallas_call(
        _mlp_body,
        grid=(3, GRID2),
        in_specs=[
            pl.BlockSpec((MC2, K, C_PAD),
                         lambda p, i: (jnp.where(p == 0, i, 0), 0, 0)),
            pl.BlockSpec((MC2, C_PAD),
                         lambda p, i: (jnp.where(p == 0, i, 0), 0)),
            pl.BlockSpec((2 * C_PAD, 128), lambda p, i: (0, 0)),
            pl.BlockSpec((128, 128), lambda p, i: (0, 0)),
            pl.BlockSpec((128, 256), lambda p, i: (0, 0)),
            pl.BlockSpec((1, 64), lambda p, i: (0, 0)),
            pl.BlockSpec((1, 64), lambda p, i: (0, 0)),
            pl.BlockSpec((1, 64), lambda p, i: (0, 0)),
            pl.BlockSpec((1, 64), lambda p, i: (0, 0)),
        ],
        out_specs=[
            pl.BlockSpec((MC2, 128), lambda p, i: (i, 0)),
            pl.BlockSpec((MC2, 128), lambda p, i: (i, 0)),
            pl.BlockSpec((2, 128), lambda p, i: (0, 0)),
        ],
        out_shape=[
            jax.ShapeDtypeStruct((B * M, 128), jnp.float32),
            jax.ShapeDtypeStruct((B * M, 128), jnp.float32),
            jax.ShapeDtypeStruct((2, 128), jnp.float32),
        ],
        scratch_shapes=[
            pltpu.VMEM((P_ROWS // 2, 128), jnp.float32),
            pltpu.VMEM((2, 64), jnp.float32),
            pltpu.VMEM((2, 64), jnp.float32),
        ],
    )(rows3d, nxpad, w0d, w1d, w2d, g0, b0, g1, b1)


def _l3_body(mx_ref, mn_ref, st_ref, g_ref, b_ref, o_ref):
    inv_p = 1.0 / P_ROWS
    mean = st_ref[0:1, :] * inv_p
    var = st_ref[1:2, :] * inv_p - mean * mean
    a = g_ref[...] * lax.rsqrt(var + EPS)
    d = b_ref[...] - mean * a
    v = jnp.maximum(mx_ref[...] * a + d, mn_ref[...] * a + d)
    o_ref[...] = jnp.maximum(v, 0.0)


def _run_l3(ymax, ymin, stats, g, b):
    return pl.pallas_call(
        _l3_body,
        grid=(B,),
        in_specs=[
            pl.BlockSpec((M, 128), lambda i: (i, 0)),
            pl.BlockSpec((M, 128), lambda i: (i, 0)),
            pl.BlockSpec((2, 128), lambda i: (0, 0)),
            pl.BlockSpec((1, 128), lambda i: (0, 0)),
            pl.BlockSpec((1, 128), lambda i: (0, 0)),
        ],
        out_specs=pl.BlockSpec((M, 128), lambda i: (i, 0)),
        out_shape=jax.ShapeDtypeStruct((B * M, 128), jnp.float32),
    )(ymax, ymin, stats, g, b)


def kernel(xyz, points, W0, g0, b0, W1, g1, b1, W2, g2, b2):
    xyzT = jnp.transpose(xyz, (2, 0, 1))

    nxyzT = _run_fps(xyzT)
    new_xyz = jnp.transpose(nxyzT, (1, 2, 0))

    nxyz8 = jnp.pad(new_xyz, ((0, 0), (0, 0), (0, 5)))
    xyzB8 = jnp.pad(jnp.transpose(xyz, (0, 2, 1)),
                    ((0, 0), (0, 5), (0, 0)))
    idx_flat = _run_ballq(nxyz8, xyzB8).reshape(-1)

    table = jnp.concatenate(
        [points, xyz, jnp.zeros((B, N, C_PAD - C_IN), jnp.float32)], axis=-1
    ).reshape(B * N, C_PAD)
    rows = _run_gather(table, idx_flat)

    nxpad = jnp.concatenate(
        [jnp.zeros((B * M, D_PTS), jnp.float32),
         new_xyz.reshape(B * M, 3),
         jnp.zeros((B * M, C_PAD - C_IN), jnp.float32)], axis=-1)

    rows3d = rows.reshape(B * M, K, C_PAD)
    w0t = jnp.pad(jnp.transpose(W0), ((0, C_PAD - C_IN), (0, 0)))
    ymax, ymin, st2 = _run_mlp(rows3d, nxpad, w0t, jnp.transpose(W1),
                               jnp.transpose(W2), g0.reshape(1, 64),
                               b0.reshape(1, 64), g1.reshape(1, 64),
                               b1.reshape(1, 64))
    pooled = _run_l3(ymax, ymin, st2, g2.reshape(1, 128), b2.reshape(1, 128))

    new_points = pooled.reshape(B, M, 128)
    return (new_xyz, new_points)
